# baseline (speedup 1.0000x reference)
import dataclasses

import jax
import jax.numpy as jnp
from jax import lax
from jax.experimental import pallas as pl
from jax.experimental.pallas import tpu as pltpu
from jax.experimental.pallas import tpu_sc as plsc

_N = 20000
_NP = 20480
_TILES = 16
_PER = _NP // _TILES
_G = _PER // 16
_MAX_OUT = 300
_OUT_PAD = 304
_IOU_THR = 0.3
_NEG_INF = float("-inf")
_BIG = 2**30


def _sc_nms(y1_hbm, x1_hbm, y2_hbm, x2_hbm, sc_hbm,
            out_hbm, shared_a, shared_b,
            ymin_v, ymax_v, xmin_v, xmax_v, area_v, sc_v,
            all_v, cand_v, keep_v,
            sem_in, sem_pub, sem_rd, sem_out):
    cid = lax.axis_index("c")
    tid = lax.axis_index("s")
    base = tid * _PER
    lane = lax.iota(jnp.int32, 16)

    @pl.when(cid == 0)
    def _body():
        pltpu.async_copy(y1_hbm.at[pl.ds(base, _PER)], ymin_v, sem_in).wait()
        pltpu.async_copy(y2_hbm.at[pl.ds(base, _PER)], ymax_v, sem_in).wait()
        pltpu.async_copy(x1_hbm.at[pl.ds(base, _PER)], xmin_v, sem_in).wait()
        pltpu.async_copy(x2_hbm.at[pl.ds(base, _PER)], xmax_v, sem_in).wait()
        pltpu.async_copy(sc_hbm.at[pl.ds(base, _PER)], sc_v, sem_in).wait()

        def prolog(g, carry):
            bmax, bidx = carry
            slc = pl.ds(g * 16, 16)
            a, b = ymin_v[slc], ymax_v[slc]
            ymin_v[slc] = jnp.minimum(a, b)
            ymax_v[slc] = jnp.maximum(a, b)
            c, d = xmin_v[slc], xmax_v[slc]
            xmin_v[slc] = jnp.minimum(c, d)
            xmax_v[slc] = jnp.maximum(c, d)
            area_v[slc] = (jnp.maximum(a, b) - jnp.minimum(a, b)) * (
                jnp.maximum(c, d) - jnp.minimum(c, d))
            s = sc_v[slc]
            upd = s > bmax
            return jnp.where(upd, s, bmax), jnp.where(upd, g, bidx)

        init = (jnp.full((16,), _NEG_INF, jnp.float32),
                jnp.zeros((16,), jnp.int32))
        bmax0, bidx0 = lax.fori_loop(0, _G, prolog, init)

        def stage(bmax, bidx):
            m = jnp.max(bmax)
            eq = bmax == m
            flat = jnp.where(eq, bidx * 16 + lane, _BIG)
            lidx = jnp.min(flat)
            gidx = base + lidx
            iv = jnp.full((16,), lidx, jnp.int32)
            row = jnp.where(lane == 0, m, 0.0)
            row = jnp.where(lane == 1, gidx.astype(jnp.float32), row)
            row = jnp.where(lane == 2, plsc.load_gather(ymin_v, [iv]), row)
            row = jnp.where(lane == 3, plsc.load_gather(ymax_v, [iv]), row)
            row = jnp.where(lane == 4, plsc.load_gather(xmin_v, [iv]), row)
            row = jnp.where(lane == 5, plsc.load_gather(xmax_v, [iv]), row)
            row = jnp.where(lane == 6, plsc.load_gather(area_v, [iv]), row)
            cand_v[...] = row

        stage(bmax0, bidx0)

        def iteration(i, tbl):
            pltpu.async_copy(cand_v, tbl.at[tid], sem_pub).wait()
            plsc.subcore_barrier()
            pltpu.async_copy(tbl, all_v, sem_rd).wait()
            col = plsc.load_gather(all_v, [lane, jnp.zeros((16,), jnp.int32)])
            gmax = jnp.max(col)
            w = plsc.all_reduce_ffs(col == gmax)
            wv = jnp.full((16,), w, jnp.int32) if w.ndim == 0 else w

            def wfield(k):
                return plsc.load_gather(all_v, [wv, jnp.full((16,), k, jnp.int32)])

            widx = wfield(1).astype(jnp.int32)
            by1 = wfield(2)
            by2 = wfield(3)
            bx1 = wfield(4)
            bx2 = wfield(5)
            ba = wfield(6)
            valid = gmax > _NEG_INF

            @pl.when(tid == 0)
            def _():
                keepvec = jnp.where(valid, widx, -1)
                plsc.store_scatter(keep_v, [jnp.full((16,), i, jnp.int32)],
                                   keepvec, mask=lane == 0)

            bad_b = ba <= 0.0

            def sweep(g, carry):
                bmax, bidx = carry
                slc = pl.ds(g * 16, 16)
                ymin, ymax = ymin_v[slc], ymax_v[slc]
                xmin, xmax = xmin_v[slc], xmax_v[slc]
                area = area_v[slc]
                s = sc_v[slc]
                yy1 = jnp.maximum(ymin, by1)
                xx1 = jnp.maximum(xmin, bx1)
                yy2 = jnp.minimum(ymax, by2)
                xx2 = jnp.minimum(xmax, bx2)
                inter = jnp.maximum(yy2 - yy1, 0.0) * jnp.maximum(xx2 - xx1, 0.0)
                denom = ba + area - inter
                iou = jnp.where(bad_b | (area <= 0.0), 0.0, inter / denom)
                gvec = base + g * 16 + lane
                supp = (iou > _IOU_THR) | (gvec == widx)
                s = jnp.where(supp, _NEG_INF, s)
                sc_v[slc] = s
                upd = s > bmax
                return jnp.where(upd, s, bmax), jnp.where(upd, g, bidx)

            bmax, bidx = lax.fori_loop(0, _G, sweep, init)
            stage(bmax, bidx)

        def pair(p, _):
            iteration(2 * p, shared_a)
            iteration(2 * p + 1, shared_b)
            return 0

        lax.fori_loop(0, _MAX_OUT // 2, pair, 0)

        @pl.when(tid == 0)
        def _():
            pltpu.async_copy(keep_v, out_hbm, sem_out).wait()


@jax.jit
def kernel(rois, cls_score):
    def prep(v, pad_val):
        return jnp.pad(v, (0, _NP - _N), constant_values=pad_val)

    y1 = prep(rois[:, 1], 0.0)
    x1 = prep(rois[:, 2], 0.0)
    y2 = prep(rois[:, 3], 0.0)
    x2 = prep(rois[:, 4], 0.0)
    sc = prep(jnp.reshape(cls_score, (-1,)), _NEG_INF)

    f32 = jnp.float32
    mesh = plsc.VectorSubcoreMesh(core_axis_name="c", subcore_axis_name="s",
                                  num_cores=2, num_subcores=16)
    cp = pltpu.CompilerParams()
    if "needs_layout_passes" in pltpu.CompilerParams.__dataclass_fields__:
        cp = dataclasses.replace(cp, needs_layout_passes=False)
    run = pl.kernel(
        _sc_nms,
        out_type=(jax.ShapeDtypeStruct((_OUT_PAD,), jnp.int32),
                  jax.ShapeDtypeStruct((16, 16), f32),
                  jax.ShapeDtypeStruct((16, 16), f32)),
        mesh=mesh,
        compiler_params=cp,
        scratch_types=[
            pltpu.VMEM((_PER,), f32),
            pltpu.VMEM((_PER,), f32),
            pltpu.VMEM((_PER,), f32),
            pltpu.VMEM((_PER,), f32),
            pltpu.VMEM((_PER,), f32),
            pltpu.VMEM((_PER,), f32),
            pltpu.VMEM((16, 16), f32),
            pltpu.VMEM((16,), f32),
            pltpu.VMEM((_OUT_PAD,), jnp.int32),
            pltpu.SemaphoreType.DMA,
            pltpu.SemaphoreType.DMA,
            pltpu.SemaphoreType.DMA,
            pltpu.SemaphoreType.DMA,
        ],
    )
    out, _tbl_a, _tbl_b = run(y1, x1, y2, x2, sc)
    return out[:_MAX_OUT]
